# Initial kernel scaffold; baseline (speedup 1.0000x reference)
import jax
import jax.numpy as jnp
from jax.experimental import pallas as pl


def kernel(x_paper, x_author, ei_cites, ei_writes, ei_rev, params, additonal_arg):
    raise NotImplementedError("write your pallas kernel here")


# speedup vs baseline: 5.1492x; 5.1492x over previous
import jax
import jax.numpy as jnp
from jax import lax
from jax.experimental import pallas as pl
from jax.experimental.pallas import tpu as pltpu
from jax.experimental.pallas import tpu_sc as plsc

N = 10000
E = 160000
D = 128
H = 32
NC = 2
NS = 16
NW = NC * NS
CHUNK = 128
KPT = 40
EPT = KPT * CHUNK
EPAD = NW * EPT
NROW = EPAD // CHUNK
NPAD = 10240
RI = NPAD // NS
BLK = 1000

_F32 = jnp.float32


def _sc_segsum(npass, with_counts):
    mesh = plsc.VectorSubcoreMesh(core_axis_name="c", subcore_axis_name="s")
    out_type = [jax.ShapeDtypeStruct((NC, NPAD, H), _F32) for _ in range(npass)]
    if with_counts:
        out_type += [jax.ShapeDtypeStruct((NC, NPAD, 16), _F32) for _ in range(npass)]
    scratch = [
        pltpu.VMEM((KPT, CHUNK), jnp.int32),
        pltpu.VMEM((KPT, CHUNK), jnp.int32),
        pltpu.VMEM((CHUNK, H), _F32),
        pltpu.VMEM((CHUNK, 16), _F32),
        pltpu.SemaphoreType.DMA,
    ]
    scratch += [pltpu.VMEM_SHARED((NPAD, H), _F32) for _ in range(npass)]
    if with_counts:
        scratch += [pltpu.VMEM_SHARED((NPAD, 16), _F32) for _ in range(npass)]

    def body(*refs):
        k = 0
        z_hbm = refs[k:k + npass]; k += npass
        src_hbm = refs[k:k + npass]; k += npass
        dst_hbm = refs[k:k + npass]; k += npass
        zeros32, zeros16, ones_hbm = refs[k:k + 3]; k += 3
        s_out = refs[k:k + npass]; k += npass
        if with_counts:
            c_out = refs[k:k + npass]; k += npass
        src_v, dst_v, rows_v, ones_v, sem = refs[k:k + 5]; k += 5
        acc = refs[k:k + npass]; k += npass
        if with_counts:
            cacc = refs[k:k + npass]; k += npass

        core = lax.axis_index("c")
        sid = lax.axis_index("s")
        wid = sid * NC + core

        pltpu.sync_copy(ones_hbm, ones_v)
        for p in range(npass):
            pltpu.sync_copy(zeros32, acc[p].at[pl.ds(sid * RI, RI)])
            if with_counts:
                pltpu.sync_copy(zeros16, cacc[p].at[pl.ds(sid * RI, RI)])
        plsc.subcore_barrier()

        for p in range(npass):
            pltpu.sync_copy(src_hbm[p].at[pl.ds(wid * KPT, KPT)], src_v)
            pltpu.sync_copy(dst_hbm[p].at[pl.ds(wid * KPT, KPT)], dst_v)

            def step(j, carry, p=p):
                pltpu.async_copy(z_hbm[p].at[src_v.at[j]], rows_v, sem).wait()
                pltpu.sync_copy(rows_v, acc[p].at[dst_v.at[j]], add=True)
                if with_counts:
                    pltpu.sync_copy(ones_v, cacc[p].at[dst_v.at[j]], add=True)
                return carry

            lax.fori_loop(0, KPT, step, 0)
        plsc.subcore_barrier()

        for p in range(npass):
            pltpu.sync_copy(acc[p].at[pl.ds(sid * RI, RI)],
                            s_out[p].at[core, pl.ds(sid * RI, RI)])
            if with_counts:
                pltpu.sync_copy(cacc[p].at[pl.ds(sid * RI, RI)],
                                c_out[p].at[core, pl.ds(sid * RI, RI)])

    return pl.kernel(body, out_type=out_type, mesh=mesh, scratch_types=scratch,
                     compiler_params=pltpu.CompilerParams(use_tc_tiling_on_sc=False),
                     name="sc_segsum%d" % npass)


_SEG3 = _sc_segsum(3, True)
_SEG2 = _sc_segsum(2, False)


def _tc1(xp, xa, wlc, wlw, wlr, wrp, wra, b1p, b1r):
    def body(xp_r, xa_r, wlc_r, wlw_r, wlr_r, wrp_r, wra_r, b1p_r, b1r_r,
             zpc_o, zaw_o, zr_o, xrp_o, xra_o):
        xp_b = xp_r[...]
        xa_b = xa_r[...]
        zpc_o[...] = jnp.dot(xp_b, wlc_r[...], preferred_element_type=_F32)
        zaw_o[...] = jnp.dot(xa_b, wlw_r[...], preferred_element_type=_F32)
        zr_o[...] = jnp.dot(xp_b, wlr_r[...], preferred_element_type=_F32)
        xrp_o[...] = jnp.dot(xp_b, wrp_r[...], preferred_element_type=_F32) + b1p_r[...]
        xra_o[...] = jnp.dot(xa_b, wra_r[...], preferred_element_type=_F32) + b1r_r[...]

    grid = (N // BLK,)
    xspec = pl.BlockSpec((BLK, D), lambda i: (i, 0))
    wspec = pl.BlockSpec((D, H), lambda i: (0, 0))
    bspec = pl.BlockSpec((1, H), lambda i: (0, 0))
    ospec = pl.BlockSpec((BLK, H), lambda i: (i, 0))
    return pl.pallas_call(
        body, grid=grid,
        in_specs=[xspec, xspec, wspec, wspec, wspec, wspec, wspec, bspec, bspec],
        out_specs=[ospec] * 5,
        out_shape=[jax.ShapeDtypeStruct((N, H), _F32)] * 5,
    )(xp, xa, wlc, wlw, wlr, wrp, wra, b1p, b1r)


def _tc2(s_c, s_w, s_r, c_c, c_w, c_r, xrp, xra, wlc2, wlw2, wr2, b2p):
    def body(sc_r, sw_r, sr_r, cc_r, cw_r, cr_r, xrp_r, xra_r,
             wlc2_r, wlw2_r, wr2_r, b2p_r, zp2_o, za2_o, xr2p_o):
        inv_c = 1.0 / jnp.maximum(cc_r[0, :, :1] + cc_r[1, :, :1], 1.0)
        inv_w = 1.0 / jnp.maximum(cw_r[0, :, :1] + cw_r[1, :, :1], 1.0)
        inv_r = 1.0 / jnp.maximum(cr_r[0, :, :1] + cr_r[1, :, :1], 1.0)
        hp = jax.nn.relu((sc_r[0] + sc_r[1]) * inv_c
                         + (sw_r[0] + sw_r[1]) * inv_w + xrp_r[...])
        ha = jax.nn.relu((sr_r[0] + sr_r[1]) * inv_r + xra_r[...])
        zp2_o[...] = jnp.dot(hp, wlc2_r[...], preferred_element_type=_F32)
        za2_o[...] = jnp.dot(ha, wlw2_r[...], preferred_element_type=_F32)
        xr2p_o[...] = jnp.dot(hp, wr2_r[...], preferred_element_type=_F32) + b2p_r[...]

    grid = (N // BLK,)
    sspec = pl.BlockSpec((NC, BLK, H), lambda i: (0, i, 0))
    cspec = pl.BlockSpec((NC, BLK, 16), lambda i: (0, i, 0))
    xspec = pl.BlockSpec((BLK, H), lambda i: (i, 0))
    wspec = pl.BlockSpec((H, H), lambda i: (0, 0))
    bspec = pl.BlockSpec((1, H), lambda i: (0, 0))
    return pl.pallas_call(
        body, grid=grid,
        in_specs=[sspec, sspec, sspec, cspec, cspec, cspec, xspec, xspec,
                  wspec, wspec, wspec, bspec],
        out_specs=[xspec] * 3,
        out_shape=[jax.ShapeDtypeStruct((N, H), _F32)] * 3,
    )(s_c, s_w, s_r, c_c, c_w, c_r, xrp, xra, wlc2, wlw2, wr2, b2p)


def _tc3(s2c, s2w, c_c, c_w, xr2p):
    def body(sc_r, sw_r, cc_r, cw_r, xr_r, out_o):
        inv_c = 1.0 / jnp.maximum(cc_r[0, :, :1] + cc_r[1, :, :1], 1.0)
        inv_w = 1.0 / jnp.maximum(cw_r[0, :, :1] + cw_r[1, :, :1], 1.0)
        out_o[...] = ((sc_r[0] + sc_r[1]) * inv_c
                      + (sw_r[0] + sw_r[1]) * inv_w + xr_r[...])

    grid = (N // BLK,)
    sspec = pl.BlockSpec((NC, BLK, H), lambda i: (0, i, 0))
    cspec = pl.BlockSpec((NC, BLK, 16), lambda i: (0, i, 0))
    xspec = pl.BlockSpec((BLK, H), lambda i: (i, 0))
    return pl.pallas_call(
        body, grid=grid,
        in_specs=[sspec, sspec, cspec, cspec, xspec],
        out_specs=xspec,
        out_shape=jax.ShapeDtypeStruct((N, H), _F32),
    )(s2c, s2w, c_c, c_w, xr2p)


def _prep_idx(ei):
    pad = EPAD - E
    src = jnp.concatenate([ei[0], jnp.zeros((pad,), jnp.int32)])
    dst = jnp.concatenate([ei[1], jnp.full((pad,), N, jnp.int32)])
    return src.reshape(NROW, CHUNK), dst.reshape(NROW, CHUNK)


def kernel(x_paper, x_author, ei_cites, ei_writes, ei_rev, params, additonal_arg):
    p = params
    src_c, dst_c = _prep_idx(ei_cites)
    src_w, dst_w = _prep_idx(ei_writes)
    src_r, dst_r = _prep_idx(ei_rev)
    zeros32 = jnp.zeros((RI, H), _F32)
    zeros16 = jnp.zeros((RI, 16), _F32)
    ones = jnp.ones((CHUNK, 16), _F32)

    b1p = (p['cites_1']['b'] + p['writes_1']['b']).reshape(1, H)
    b1r = p['rev_1']['b'].reshape(1, H)
    wrp = p['cites_1']['Wr'] + p['writes_1']['Wr']
    b2p = (p['cites_2']['b'] + p['writes_2']['b']).reshape(1, H)
    wr2 = p['cites_2']['Wr'] + p['writes_2']['Wr']

    zpc, zaw, zr, xrp, xra = _tc1(
        x_paper, x_author, p['cites_1']['Wl'], p['writes_1']['Wl'],
        p['rev_1']['Wl'], wrp, p['rev_1']['Wr'], b1p, b1r)

    s_c, s_w, s_r, c_c, c_w, c_r = _SEG3(
        zpc, zaw, zr, src_c, src_w, src_r, dst_c, dst_w, dst_r,
        zeros32, zeros16, ones)

    zp2, za2, xr2p = _tc2(s_c, s_w, s_r, c_c, c_w, c_r, xrp, xra,
                          p['cites_2']['Wl'], p['writes_2']['Wl'], wr2, b2p)

    s2c, s2w = _SEG2(zp2, za2, src_c, src_w, dst_c, dst_w,
                     zeros32, zeros16, ones)

    return _tc3(s2c, s2w, c_c, c_w, xr2p)


# speedup vs baseline: 5.7715x; 1.1209x over previous
import jax
import jax.numpy as jnp
from jax import lax
from jax.experimental import pallas as pl
from jax.experimental.pallas import tpu as pltpu
from jax.experimental.pallas import tpu_sc as plsc

N = 10000
E = 160000
D = 128
H = 32
NC = 2
NS = 16
NW = NC * NS
CHUNK = 512
KPT = 10
EPT = KPT * CHUNK
EPAD = NW * EPT
NROW = EPAD // CHUNK
NPAD = 10240
RI = NPAD // NS
BLK = 1000

_F32 = jnp.float32


def _sc_segsum(npass, with_counts):
    mesh = plsc.VectorSubcoreMesh(core_axis_name="c", subcore_axis_name="s")
    out_type = [jax.ShapeDtypeStruct((NC, NPAD, H), _F32) for _ in range(npass)]
    if with_counts:
        out_type += [jax.ShapeDtypeStruct((NC, NPAD, 16), _F32) for _ in range(npass)]
    scratch = [
        pltpu.VMEM((KPT, CHUNK), jnp.int32),
        pltpu.VMEM((KPT, CHUNK), jnp.int32),
        pltpu.VMEM((CHUNK, H), _F32),
        pltpu.VMEM((CHUNK, 16), _F32),
        pltpu.SemaphoreType.DMA,
    ]
    scratch += [pltpu.VMEM_SHARED((NPAD, H), _F32) for _ in range(npass)]
    if with_counts:
        scratch += [pltpu.VMEM_SHARED((NPAD, 16), _F32) for _ in range(npass)]

    def body(*refs):
        k = 0
        z_hbm = refs[k:k + npass]; k += npass
        src_hbm = refs[k:k + npass]; k += npass
        dst_hbm = refs[k:k + npass]; k += npass
        zeros32, zeros16, ones_hbm = refs[k:k + 3]; k += 3
        s_out = refs[k:k + npass]; k += npass
        if with_counts:
            c_out = refs[k:k + npass]; k += npass
        src_v, dst_v, rows_v, ones_v, sem = refs[k:k + 5]; k += 5
        acc = refs[k:k + npass]; k += npass
        if with_counts:
            cacc = refs[k:k + npass]; k += npass

        core = lax.axis_index("c")
        sid = lax.axis_index("s")
        wid = sid * NC + core

        pltpu.sync_copy(ones_hbm, ones_v)
        for p in range(npass):
            pltpu.sync_copy(zeros32, acc[p].at[pl.ds(sid * RI, RI)])
            if with_counts:
                pltpu.sync_copy(zeros16, cacc[p].at[pl.ds(sid * RI, RI)])
        plsc.subcore_barrier()

        for p in range(npass):
            pltpu.sync_copy(src_hbm[p].at[pl.ds(wid * KPT, KPT)], src_v)
            pltpu.sync_copy(dst_hbm[p].at[pl.ds(wid * KPT, KPT)], dst_v)

            def step(j, carry, p=p):
                pltpu.async_copy(z_hbm[p].at[src_v.at[j]], rows_v, sem).wait()
                pltpu.sync_copy(rows_v, acc[p].at[dst_v.at[j]], add=True)
                if with_counts:
                    pltpu.sync_copy(ones_v, cacc[p].at[dst_v.at[j]], add=True)
                return carry

            lax.fori_loop(0, KPT, step, 0)
        plsc.subcore_barrier()

        for p in range(npass):
            pltpu.sync_copy(acc[p].at[pl.ds(sid * RI, RI)],
                            s_out[p].at[core, pl.ds(sid * RI, RI)])
            if with_counts:
                pltpu.sync_copy(cacc[p].at[pl.ds(sid * RI, RI)],
                                c_out[p].at[core, pl.ds(sid * RI, RI)])

    return pl.kernel(body, out_type=out_type, mesh=mesh, scratch_types=scratch,
                     compiler_params=pltpu.CompilerParams(use_tc_tiling_on_sc=False),
                     name="sc_segsum%d" % npass)


_SEG3 = _sc_segsum(3, True)
_SEG2 = _sc_segsum(2, False)


def _tc1(xp, xa, wlc, wlw, wlr, wrp, wra, b1p, b1r):
    def body(xp_r, xa_r, wlc_r, wlw_r, wlr_r, wrp_r, wra_r, b1p_r, b1r_r,
             zpc_o, zaw_o, zr_o, xrp_o, xra_o):
        xp_b = xp_r[...]
        xa_b = xa_r[...]
        zpc_o[...] = jnp.dot(xp_b, wlc_r[...], preferred_element_type=_F32)
        zaw_o[...] = jnp.dot(xa_b, wlw_r[...], preferred_element_type=_F32)
        zr_o[...] = jnp.dot(xp_b, wlr_r[...], preferred_element_type=_F32)
        xrp_o[...] = jnp.dot(xp_b, wrp_r[...], preferred_element_type=_F32) + b1p_r[...]
        xra_o[...] = jnp.dot(xa_b, wra_r[...], preferred_element_type=_F32) + b1r_r[...]

    grid = (N // BLK,)
    xspec = pl.BlockSpec((BLK, D), lambda i: (i, 0))
    wspec = pl.BlockSpec((D, H), lambda i: (0, 0))
    bspec = pl.BlockSpec((1, H), lambda i: (0, 0))
    ospec = pl.BlockSpec((BLK, H), lambda i: (i, 0))
    return pl.pallas_call(
        body, grid=grid,
        in_specs=[xspec, xspec, wspec, wspec, wspec, wspec, wspec, bspec, bspec],
        out_specs=[ospec] * 5,
        out_shape=[jax.ShapeDtypeStruct((N, H), _F32)] * 5,
    )(xp, xa, wlc, wlw, wlr, wrp, wra, b1p, b1r)


def _tc2(s_c, s_w, s_r, c_c, c_w, c_r, xrp, xra, wlc2, wlw2, wr2, b2p):
    def body(sc_r, sw_r, sr_r, cc_r, cw_r, cr_r, xrp_r, xra_r,
             wlc2_r, wlw2_r, wr2_r, b2p_r, zp2_o, za2_o, xr2p_o):
        inv_c = 1.0 / jnp.maximum(cc_r[0, :, :1] + cc_r[1, :, :1], 1.0)
        inv_w = 1.0 / jnp.maximum(cw_r[0, :, :1] + cw_r[1, :, :1], 1.0)
        inv_r = 1.0 / jnp.maximum(cr_r[0, :, :1] + cr_r[1, :, :1], 1.0)
        hp = jax.nn.relu((sc_r[0] + sc_r[1]) * inv_c
                         + (sw_r[0] + sw_r[1]) * inv_w + xrp_r[...])
        ha = jax.nn.relu((sr_r[0] + sr_r[1]) * inv_r + xra_r[...])
        zp2_o[...] = jnp.dot(hp, wlc2_r[...], preferred_element_type=_F32)
        za2_o[...] = jnp.dot(ha, wlw2_r[...], preferred_element_type=_F32)
        xr2p_o[...] = jnp.dot(hp, wr2_r[...], preferred_element_type=_F32) + b2p_r[...]

    grid = (N // BLK,)
    sspec = pl.BlockSpec((NC, BLK, H), lambda i: (0, i, 0))
    cspec = pl.BlockSpec((NC, BLK, 16), lambda i: (0, i, 0))
    xspec = pl.BlockSpec((BLK, H), lambda i: (i, 0))
    wspec = pl.BlockSpec((H, H), lambda i: (0, 0))
    bspec = pl.BlockSpec((1, H), lambda i: (0, 0))
    return pl.pallas_call(
        body, grid=grid,
        in_specs=[sspec, sspec, sspec, cspec, cspec, cspec, xspec, xspec,
                  wspec, wspec, wspec, bspec],
        out_specs=[xspec] * 3,
        out_shape=[jax.ShapeDtypeStruct((N, H), _F32)] * 3,
    )(s_c, s_w, s_r, c_c, c_w, c_r, xrp, xra, wlc2, wlw2, wr2, b2p)


def _tc3(s2c, s2w, c_c, c_w, xr2p):
    def body(sc_r, sw_r, cc_r, cw_r, xr_r, out_o):
        inv_c = 1.0 / jnp.maximum(cc_r[0, :, :1] + cc_r[1, :, :1], 1.0)
        inv_w = 1.0 / jnp.maximum(cw_r[0, :, :1] + cw_r[1, :, :1], 1.0)
        out_o[...] = ((sc_r[0] + sc_r[1]) * inv_c
                      + (sw_r[0] + sw_r[1]) * inv_w + xr_r[...])

    grid = (N // BLK,)
    sspec = pl.BlockSpec((NC, BLK, H), lambda i: (0, i, 0))
    cspec = pl.BlockSpec((NC, BLK, 16), lambda i: (0, i, 0))
    xspec = pl.BlockSpec((BLK, H), lambda i: (i, 0))
    return pl.pallas_call(
        body, grid=grid,
        in_specs=[sspec, sspec, cspec, cspec, xspec],
        out_specs=xspec,
        out_shape=jax.ShapeDtypeStruct((N, H), _F32),
    )(s2c, s2w, c_c, c_w, xr2p)


def _prep_idx(ei):
    pad = EPAD - E
    src = jnp.concatenate([ei[0], jnp.zeros((pad,), jnp.int32)])
    dst = jnp.concatenate([ei[1], jnp.full((pad,), N, jnp.int32)])
    return src.reshape(NROW, CHUNK), dst.reshape(NROW, CHUNK)


def kernel(x_paper, x_author, ei_cites, ei_writes, ei_rev, params, additonal_arg):
    p = params
    src_c, dst_c = _prep_idx(ei_cites)
    src_w, dst_w = _prep_idx(ei_writes)
    src_r, dst_r = _prep_idx(ei_rev)
    zeros32 = jnp.zeros((RI, H), _F32)
    zeros16 = jnp.zeros((RI, 16), _F32)
    ones = jnp.ones((CHUNK, 16), _F32)

    b1p = (p['cites_1']['b'] + p['writes_1']['b']).reshape(1, H)
    b1r = p['rev_1']['b'].reshape(1, H)
    wrp = p['cites_1']['Wr'] + p['writes_1']['Wr']
    b2p = (p['cites_2']['b'] + p['writes_2']['b']).reshape(1, H)
    wr2 = p['cites_2']['Wr'] + p['writes_2']['Wr']

    zpc, zaw, zr, xrp, xra = _tc1(
        x_paper, x_author, p['cites_1']['Wl'], p['writes_1']['Wl'],
        p['rev_1']['Wl'], wrp, p['rev_1']['Wr'], b1p, b1r)

    s_c, s_w, s_r, c_c, c_w, c_r = _SEG3(
        zpc, zaw, zr, src_c, src_w, src_r, dst_c, dst_w, dst_r,
        zeros32, zeros16, ones)

    zp2, za2, xr2p = _tc2(s_c, s_w, s_r, c_c, c_w, c_r, xrp, xra,
                          p['cites_2']['Wl'], p['writes_2']['Wl'], wr2, b2p)

    s2c, s2w = _SEG2(zp2, za2, src_c, src_w, dst_c, dst_w,
                     zeros32, zeros16, ones)

    return _tc3(s2c, s2w, c_c, c_w, xr2p)
